# initial kernel scaffold (unmeasured)
import jax
import jax.numpy as jnp
from jax import lax
from jax.experimental import pallas as pl
from jax.experimental.pallas import tpu as pltpu


def kernel(
    x,
):
    def body(*refs):
        pass

    out_shape = jax.ShapeDtypeStruct(..., jnp.float32)
    return pl.pallas_call(body, out_shape=out_shape)(...)


# baseline (device time: 205821 ns/iter reference)
import functools

import jax
import jax.numpy as jnp
from jax import lax
from jax.experimental import pallas as pl
from jax.experimental.pallas import tpu as pltpu


def kernel(x):
    _, m, n2 = x.shape
    n = n2 // 2

    def body(x_ref, out_ref, comm_ref, send_sem, recv_sem, local_sem):
        my_x = lax.axis_index("x")
        my_y = lax.axis_index("y")
        y_peer = (my_x, 1 - my_y)

        barrier = pltpu.get_barrier_semaphore()
        pl.semaphore_signal(
            barrier, inc=1, device_id=y_peer, device_id_type=pl.DeviceIdType.MESH
        )
        pl.semaphore_wait(barrier, 1)

        def exchange(peer_col0, my_col0):
            rdma = pltpu.make_async_remote_copy(
                src_ref=x_ref.at[0, :, pl.ds(peer_col0, n)],
                dst_ref=comm_ref,
                send_sem=send_sem,
                recv_sem=recv_sem,
                device_id=y_peer,
                device_id_type=pl.DeviceIdType.MESH,
            )
            rdma.start()
            local = pltpu.make_async_copy(
                x_ref.at[0, :, pl.ds(my_col0, n)], out_ref, local_sem
            )
            local.start()
            local.wait()
            rdma.wait()
            out_ref[:, :] = out_ref[:, :] + comm_ref[:, :]

        @pl.when(my_y == 0)
        def _():
            exchange(n, 0)

        @pl.when(my_y == 1)
        def _():
            exchange(0, n)

        @functools.partial(pl.run_scoped, sem=pltpu.SemaphoreType.REGULAR)
        def _(sem):
            pl.semaphore_signal(
                sem, inc=1, device_id=y_peer, device_id_type=pl.DeviceIdType.MESH
            )
            pl.semaphore_wait(sem, 1)

    return pl.pallas_call(
        body,
        out_shape=jax.ShapeDtypeStruct((m, n), x.dtype),
        in_specs=[pl.BlockSpec(memory_space=pl.ANY)],
        out_specs=pl.BlockSpec(memory_space=pltpu.VMEM),
        scratch_shapes=[
            pltpu.VMEM((m, n), x.dtype),
            pltpu.SemaphoreType.DMA,
            pltpu.SemaphoreType.DMA,
            pltpu.SemaphoreType.DMA,
        ],
        compiler_params=pltpu.CompilerParams(
            collective_id=0, vmem_limit_bytes=40 * 1024 * 1024
        ),
    )(x)


# device time: 121351 ns/iter; 1.6961x vs baseline; 1.6961x over previous
import jax
import jax.numpy as jnp
from jax import lax
from jax.experimental import pallas as pl
from jax.experimental.pallas import tpu as pltpu

C = 16


def kernel(x):
    _, m, n2 = x.shape
    n = n2 // 2
    half_m = m // 2
    ch = half_m // C

    def body(x_ref, out_ref, comm_ref, y_send, y_recv, x_send, x_recv, local_sems):
        my_x = lax.axis_index("x")
        my_y = lax.axis_index("y")
        y_peer = (my_x, 1 - my_y)
        x_peer = (1 - my_x, my_y)

        barrier = pltpu.get_barrier_semaphore()
        for p in (y_peer, x_peer):
            pl.semaphore_signal(
                barrier, inc=1, device_id=p, device_id_type=pl.DeviceIdType.MESH
            )
        pl.semaphore_wait(barrier, 2)

        row0 = my_x * half_m
        my_col0 = my_y * n
        peer_col0 = (1 - my_y) * n

        y_rdmas = []
        x_rdmas = []
        local_dmas = []

        for c in range(C):
            r = row0 + c * ch
            dma = pltpu.make_async_copy(
                x_ref.at[0, pl.ds(r, ch), pl.ds(my_col0, n)],
                out_ref.at[pl.ds(r, ch), :],
                local_sems.at[c],
            )
            dma.start()
            local_dmas.append(dma)
            rdma = pltpu.make_async_remote_copy(
                src_ref=x_ref.at[0, pl.ds(r, ch), pl.ds(peer_col0, n)],
                dst_ref=comm_ref.at[c],
                send_sem=y_send.at[c],
                recv_sem=y_recv.at[c],
                device_id=y_peer,
                device_id_type=pl.DeviceIdType.MESH,
            )
            rdma.start()
            y_rdmas.append(rdma)

        for c in range(C):
            r = row0 + c * ch
            y_rdmas[c].wait_recv()
            local_dmas[c].wait()
            out_ref[pl.ds(r, ch), :] = out_ref[pl.ds(r, ch), :] + comm_ref[c]
            rdma = pltpu.make_async_remote_copy(
                src_ref=out_ref.at[pl.ds(r, ch), :],
                dst_ref=out_ref.at[pl.ds(r, ch), :],
                send_sem=x_send.at[c],
                recv_sem=x_recv.at[c],
                device_id=x_peer,
                device_id_type=pl.DeviceIdType.MESH,
            )
            rdma.start()
            x_rdmas.append(rdma)

        for c in range(C):
            y_rdmas[c].wait_send()
            x_rdmas[c].wait_send()
            x_rdmas[c].wait_recv()

    return pl.pallas_call(
        body,
        out_shape=jax.ShapeDtypeStruct((m, n), x.dtype),
        in_specs=[pl.BlockSpec(memory_space=pl.ANY)],
        out_specs=pl.BlockSpec(memory_space=pltpu.VMEM),
        scratch_shapes=[
            pltpu.VMEM((C, ch, n), x.dtype),
            pltpu.SemaphoreType.DMA((C,)),
            pltpu.SemaphoreType.DMA((C,)),
            pltpu.SemaphoreType.DMA((C,)),
            pltpu.SemaphoreType.DMA((C,)),
            pltpu.SemaphoreType.DMA((C,)),
        ],
        compiler_params=pltpu.CompilerParams(
            collective_id=0, vmem_limit_bytes=40 * 1024 * 1024
        ),
    )(x)
